# Initial kernel scaffold; baseline (speedup 1.0000x reference)
import jax
import jax.numpy as jnp
from jax.experimental import pallas as pl


def kernel(x, weight, bias):
    raise NotImplementedError("write your pallas kernel here")


# speedup vs baseline: 1.1420x; 1.1420x over previous
import functools
import numpy as np
import jax
import jax.numpy as jnp
from jax import lax
from jax.experimental import pallas as pl
from jax.experimental.pallas import tpu as pltpu

_NUM_EMBEDS = 16


def _pool_grid(num_embeds):
    if num_embeds in (1, 2, 3, 5, 7):
        return (num_embeds, 1)
    return {4: (2, 2), 6: (3, 2), 8: (4, 2), 9: (3, 3),
            16: (4, 4), 25: (5, 5), 36: (6, 6)}[num_embeds]


def _pool_matrix_T(H, W, gh, gw):
    PT = np.zeros((H * W, gh * gw), dtype=np.float32)
    for i in range(gh):
        h0 = (i * H) // gh
        h1 = -(-((i + 1) * H) // gh)
        for j in range(gw):
            w0 = (j * W) // gw
            w1 = -(-((j + 1) * W) // gw)
            cnt = float((h1 - h0) * (w1 - w0))
            for hh in range(h0, h1):
                for ww in range(w0, w1):
                    PT[hh * W + ww, i * gw + j] = 1.0 / cnt
    return PT


def _fused_kernel(tb, x_ref, pt_ref, wt_ref, b_ref, o_ref):
    pooled = None
    for b in range(tb):
        d = lax.dot_general(
            x_ref[b], pt_ref[b],
            dimension_numbers=(((1,), (0,)), ((), ())),
            preferred_element_type=jnp.float32)
        pooled = d if pooled is None else pooled + d
    y = lax.dot_general(
        pooled, wt_ref[...],
        dimension_numbers=(((0,), (0,)), ((), ())),
        preferred_element_type=jnp.float32)
    o_ref[...] = (y + b_ref[...]).astype(o_ref.dtype)


def kernel(x, weight, bias):
    B, C, H, W = x.shape
    N = weight.shape[0]
    P = _NUM_EMBEDS
    gh, gw = _pool_grid(P)
    HW = H * W

    tb = 8 if B % 8 == 0 else 1
    grid_b = B // tb

    pt = _pool_matrix_T(H, W, gh, gw)
    ptb_np = np.zeros((tb, HW, tb * P), np.float32)
    for b in range(tb):
        ptb_np[b, :, b * P:(b + 1) * P] = pt
    ptb = jnp.asarray(ptb_np)

    x3 = x.reshape(B, C, HW)
    wt = weight.T
    b2 = bias.reshape(1, N)

    cost = pl.CostEstimate(
        flops=2 * B * (C * HW * P + P * C * N),
        transcendentals=0,
        bytes_accessed=4 * (B * C * HW + N * C + N + B * P * N),
    )

    out = pl.pallas_call(
        functools.partial(_fused_kernel, tb),
        out_shape=jax.ShapeDtypeStruct((B * P, N), x.dtype),
        grid=(grid_b,),
        in_specs=[
            pl.BlockSpec((tb, C, HW), lambda i: (i, 0, 0)),
            pl.BlockSpec((tb, HW, tb * P), lambda i: (0, 0, 0)),
            pl.BlockSpec((C, N), lambda i: (0, 0)),
            pl.BlockSpec((1, N), lambda i: (0, 0)),
        ],
        out_specs=pl.BlockSpec((tb * P, N), lambda i: (i, 0)),
        compiler_params=pltpu.CompilerParams(
            dimension_semantics=("parallel",),
            vmem_limit_bytes=64 * 1024 * 1024,
        ),
        cost_estimate=cost,
    )(x3, ptb, wt, b2)

    return out.reshape(B, P, N)


# speedup vs baseline: 3.9109x; 3.4245x over previous
import functools
import numpy as np
import jax
import jax.numpy as jnp
from jax import lax
from jax.experimental import pallas as pl
from jax.experimental.pallas import tpu as pltpu

_NUM_EMBEDS = 16


def _pool_grid(num_embeds):
    if num_embeds in (1, 2, 3, 5, 7):
        return (num_embeds, 1)
    return {4: (2, 2), 6: (3, 2), 8: (4, 2), 9: (3, 3),
            16: (4, 4), 25: (5, 5), 36: (6, 6)}[num_embeds]


def _pool_matrix(H, W, gh, gw):
    P = np.zeros((gh * gw, H * W), dtype=np.float32)
    for i in range(gh):
        h0 = (i * H) // gh
        h1 = -(-((i + 1) * H) // gh)
        for j in range(gw):
            w0 = (j * W) // gw
            w1 = -(-((j + 1) * W) // gw)
            cnt = float((h1 - h0) * (w1 - w0))
            for hh in range(h0, h1):
                for ww in range(w0, w1):
                    P[i * gw + j, hh * W + ww] = 1.0 / cnt
    return P


def _fused_kernel(tb, x_ref, pb_ref, wt_ref, b_ref, o_ref):
    tb_hw = pb_ref.shape[1]
    c = x_ref.shape[2]
    xflat = x_ref[...].reshape(tb_hw, c)
    pooled = lax.dot_general(
        pb_ref[...], xflat,
        dimension_numbers=(((1,), (0,)), ((), ())),
        preferred_element_type=jnp.float32)
    y = lax.dot_general(
        pooled, wt_ref[...],
        dimension_numbers=(((1,), (0,)), ((), ())),
        preferred_element_type=jnp.float32)
    o_ref[...] = (y + b_ref[...]).astype(o_ref.dtype)


def kernel(x, weight, bias):
    B, C, H, W = x.shape
    N = weight.shape[0]
    P = _NUM_EMBEDS
    gh, gw = _pool_grid(P)
    HW = H * W

    tb = 8 if B % 8 == 0 else 1
    grid_b = B // tb

    pmat = _pool_matrix(H, W, gh, gw)
    pblk_np = np.zeros((tb * P, tb * HW), np.float32)
    for b in range(tb):
        pblk_np[b * P:(b + 1) * P, b * HW:(b + 1) * HW] = pmat
    pblk = jnp.asarray(pblk_np)

    x_hwc = jnp.transpose(x, (0, 2, 3, 1)).reshape(B, HW, C)
    wt = weight.T
    b2 = bias.reshape(1, N)

    cost = pl.CostEstimate(
        flops=2 * B * (P * HW * C + P * C * N),
        transcendentals=0,
        bytes_accessed=4 * (B * C * HW + N * C + N + B * P * N),
    )

    out = pl.pallas_call(
        functools.partial(_fused_kernel, tb),
        out_shape=jax.ShapeDtypeStruct((B * P, N), x.dtype),
        grid=(grid_b,),
        in_specs=[
            pl.BlockSpec((tb, HW, C), lambda i: (i, 0, 0)),
            pl.BlockSpec((tb * P, tb * HW), lambda i: (0, 0)),
            pl.BlockSpec((C, N), lambda i: (0, 0)),
            pl.BlockSpec((1, N), lambda i: (0, 0)),
        ],
        out_specs=pl.BlockSpec((tb * P, N), lambda i: (i, 0)),
        compiler_params=pltpu.CompilerParams(
            dimension_semantics=("parallel",),
            vmem_limit_bytes=64 * 1024 * 1024,
        ),
        cost_estimate=cost,
    )(x_hwc, pblk, wt, b2)

    return out.reshape(B, P, N)


# speedup vs baseline: 4.2207x; 1.0792x over previous
import functools
import numpy as np
import jax
import jax.numpy as jnp
from jax import lax
from jax.experimental import pallas as pl
from jax.experimental.pallas import tpu as pltpu

_NUM_EMBEDS = 16


def _pool_grid(num_embeds):
    if num_embeds in (1, 2, 3, 5, 7):
        return (num_embeds, 1)
    return {4: (2, 2), 6: (3, 2), 8: (4, 2), 9: (3, 3),
            16: (4, 4), 25: (5, 5), 36: (6, 6)}[num_embeds]


def _pool_matrix(H, W, gh, gw):
    P = np.zeros((gh * gw, H * W), dtype=np.float32)
    for i in range(gh):
        h0 = (i * H) // gh
        h1 = -(-((i + 1) * H) // gh)
        for j in range(gw):
            w0 = (j * W) // gw
            w1 = -(-((j + 1) * W) // gw)
            cnt = float((h1 - h0) * (w1 - w0))
            for hh in range(h0, h1):
                for ww in range(w0, w1):
                    P[i * gw + j, hh * W + ww] = 1.0 / cnt
    return P


def _fused_kernel(tb, x_ref, pb_ref, w_ref, b_ref, o_ref):
    tb_hw = pb_ref.shape[1]
    c = x_ref.shape[2]
    xflat = x_ref[...].reshape(tb_hw, c)
    pooled = lax.dot_general(
        pb_ref[...], xflat,
        dimension_numbers=(((1,), (0,)), ((), ())),
        preferred_element_type=jnp.float32)
    y = lax.dot_general(
        pooled, w_ref[...],
        dimension_numbers=(((1,), (1,)), ((), ())),
        preferred_element_type=jnp.float32)
    y = (y + b_ref[...]).astype(o_ref.dtype)
    o_ref[...] = y.reshape(o_ref.shape)


def kernel(x, weight, bias):
    B, C, H, W = x.shape
    N = weight.shape[0]
    P = _NUM_EMBEDS
    gh, gw = _pool_grid(P)
    HW = H * W

    tb = 8 if B % 8 == 0 else 1
    grid_b = B // tb

    pmat = _pool_matrix(H, W, gh, gw)
    pblk_np = np.zeros((tb * P, tb * HW), np.float32)
    for b in range(tb):
        pblk_np[b * P:(b + 1) * P, b * HW:(b + 1) * HW] = pmat
    pblk = jnp.asarray(pblk_np)

    x_hwc = jnp.transpose(x, (0, 2, 3, 1)).reshape(B, HW, C)
    b2 = bias.reshape(1, N)

    cost = pl.CostEstimate(
        flops=2 * B * (P * HW * C + P * C * N),
        transcendentals=0,
        bytes_accessed=4 * (B * C * HW + N * C + N + B * P * N),
    )

    out = pl.pallas_call(
        functools.partial(_fused_kernel, tb),
        out_shape=jax.ShapeDtypeStruct((B, P, N), x.dtype),
        grid=(grid_b,),
        in_specs=[
            pl.BlockSpec((tb, HW, C), lambda i: (i, 0, 0)),
            pl.BlockSpec((tb * P, tb * HW), lambda i: (0, 0)),
            pl.BlockSpec((N, C), lambda i: (0, 0)),
            pl.BlockSpec((1, N), lambda i: (0, 0)),
        ],
        out_specs=pl.BlockSpec((tb, P, N), lambda i: (i, 0, 0)),
        compiler_params=pltpu.CompilerParams(
            dimension_semantics=("parallel",),
            vmem_limit_bytes=64 * 1024 * 1024,
        ),
        cost_estimate=cost,
    )(x_hwc, pblk, weight, b2)

    return out
